# Initial kernel scaffold; baseline (speedup 1.0000x reference)
import jax
import jax.numpy as jnp
from jax.experimental import pallas as pl


def kernel(x, u, params, edge_index, batch):
    raise NotImplementedError("write your pallas kernel here")


# speedup vs baseline: 1.0015x; 1.0015x over previous
import functools

import jax
import jax.numpy as jnp
from jax.experimental import pallas as pl


def _layer_norm(x, g, b, eps=1e-5):
    m = jnp.mean(x, axis=-1, keepdims=True)
    v = jnp.var(x, axis=-1, keepdims=True)
    return (x - m) / jnp.sqrt(v + eps) * g + b


def _gelu(x):
    return 0.5 * x * (1.0 + jax.lax.erf(x * 0.7071067811865476))


def _gat_conv(h_in, src, dst, n, W, a_src, a_dst, bias):
    heads = a_src.shape[0]
    C = W.shape[1] // heads
    h = (h_in @ W).reshape(n, heads, C)
    al_s = jnp.sum(h * a_src[None], axis=-1)
    al_d = jnp.sum(h * a_dst[None], axis=-1)
    e = jax.nn.leaky_relu(al_s[src] + al_d[dst], negative_slope=0.2)
    m = jax.ops.segment_max(e, dst, num_segments=n)
    ex = jnp.exp(e - m[dst])
    s = jax.ops.segment_sum(ex, dst, num_segments=n)
    alpha = ex / (s[dst] + 1e-16)
    out = jax.ops.segment_sum(h[src] * alpha[:, :, None], dst, num_segments=n)
    return out.reshape(n, heads * C) + bias


def _gat_block(h, src, dst, n, p):
    res = h @ p['rw'] + p['rb'] if 'rw' in p else h
    hh = _gat_conv(h, src, dst, n, p['W'], p['as'], p['ad'], p['b'])
    hh = _layer_norm(hh, p['ng'], p['nb'])
    return _gelu(hh + res)


def _mlp_kernel(gfeat_ref, u_ref, gw1_ref, gb1_ref, gg1_ref, gbe1_ref,
                gw2_ref, gb2_ref, gg2_ref, gbe2_ref,
                fw1_ref, fb1_ref, fg1_ref, fbe1_ref,
                fw2_ref, fb2_ref, fg2_ref, fbe2_ref,
                cw1_ref, cb1_ref, cw2_ref, cb2_ref, out_ref):
    u = u_ref[...]
    g = _gelu(_layer_norm(u @ gw1_ref[...] + gb1_ref[...], gg1_ref[...], gbe1_ref[...]))
    g = _gelu(_layer_norm(g @ gw2_ref[...] + gb2_ref[...], gg2_ref[...], gbe2_ref[...]))
    c = jnp.concatenate([gfeat_ref[...], g], axis=1)
    c = _gelu(_layer_norm(c @ fw1_ref[...] + fb1_ref[...], fg1_ref[...], fbe1_ref[...]))
    c = _gelu(_layer_norm(c @ fw2_ref[...] + fb2_ref[...], fg2_ref[...], fbe2_ref[...]))
    c = _gelu(c @ cw1_ref[...] + cb1_ref[...])
    out_ref[...] = c @ cw2_ref[...] + cb2_ref[...]


def kernel(x, u, params, edge_index, batch):
    n = x.shape[0]
    b = u.shape[0]
    loop = jnp.arange(n)
    src = jnp.concatenate([edge_index[0], loop])
    dst = jnp.concatenate([edge_index[1], loop])
    p = params
    h = _gelu(_layer_norm(x @ p['ne_w'] + p['ne_b'], p['ne_g'], p['ne_be']))
    for name in ('g1', 'g2', 'g3', 'g4'):
        h = _gat_block(h, src, dst, n, p[name])
    counts = jax.ops.segment_sum(jnp.ones((n,), jnp.float32), batch, num_segments=b)
    gsum = jax.ops.segment_sum(h, batch, num_segments=b)
    gmean = gsum / jnp.maximum(counts, 1.0)[:, None]
    gmax = jax.ops.segment_max(h, batch, num_segments=b)
    gfeat = jnp.concatenate([gmean, gmax, gsum / 10.0], axis=1)

    out = pl.pallas_call(
        _mlp_kernel,
        out_shape=jax.ShapeDtypeStruct((b, 1), jnp.float32),
    )(gfeat, u,
      p['ge_w1'], p['ge_b1'], p['ge_g1'], p['ge_be1'],
      p['ge_w2'], p['ge_b2'], p['ge_g2'], p['ge_be2'],
      p['fu_w1'], p['fu_b1'], p['fu_g1'], p['fu_be1'],
      p['fu_w2'], p['fu_b2'], p['fu_g2'], p['fu_be2'],
      p['cl_w1'], p['cl_b1'], p['cl_w2'], p['cl_b2'])
    return out[:, 0]


# speedup vs baseline: 32.1867x; 32.1396x over previous
import functools

import jax
import jax.numpy as jnp
from jax import lax
from jax.experimental import pallas as pl
from jax.experimental.pallas import tpu as pltpu
from jax.experimental.pallas import tpu_sc as plsc

N = 50000
E = 800000
HEADS = 4

_NC = 2
_NS = 16


def _iota16():
    return lax.iota(jnp.int32, 16)


def _gelu(x):
    return 0.5 * x * (1.0 + lax.erf(x * 0.7071067811865476))


def _layer_norm(x, g, b, eps=1e-5):
    m = jnp.mean(x, axis=-1, keepdims=True)
    v = jnp.var(x, axis=-1, keepdims=True)
    return (x - m) / jnp.sqrt(v + eps) * g + b


_P1_K = 1000
_P1_CHUNKS = E // (_NC * _NS * _P1_K)


def _make_phase1():
    K = _P1_K

    def body(als_hbm, ald_hbm, src_hbm, dst_hbm, zeros16_hbm, ex_hbm, s_hbm,
             idxs_v, idxd_v, sbuf_v, dbuf_v, exrow_v, exrow16_v, merge_v,
             s_spmem):
        c = lax.axis_index("c")
        t = lax.axis_index("s")
        it = _iota16()

        @pl.when(t < 10)
        def _():
            pltpu.sync_copy(zeros16_hbm.at[pl.ds(t * 5000, 5000)],
                            s_spmem.at[pl.ds(t * 5000, 5000)])

        plsc.subcore_barrier()

        per_tec = E // (_NC * _NS)

        def chunk(i, carry):
            base = (c * _NS + t) * per_tec + i * K
            pltpu.sync_copy(src_hbm.at[pl.ds(base, K)], idxs_v)
            pltpu.sync_copy(dst_hbm.at[pl.ds(base, K)], idxd_v)
            pltpu.sync_copy(als_hbm.at[idxs_v], sbuf_v)
            pltpu.sync_copy(ald_hbm.at[idxd_v], dbuf_v)

            def group(q, carry2):
                for jj in range(4):
                    j = 4 * q + jj
                    e = sbuf_v[j] + dbuf_v[j]
                    plsc.store_scatter(merge_v, [(it % 4) + 4 * jj], e,
                                       mask=it < 4)
                m = merge_v[...]
                m = jnp.where(m > 0, m, 0.2 * m)
                exv = jnp.exp(m)
                plsc.store_scatter(exrow_v, [4 * q + it // 4, it % 4], exv)
                for jj in range(4):
                    plsc.store_scatter(
                        exrow16_v,
                        [jnp.full((16,), 4 * q + jj, jnp.int32),
                         (it - 4 * jj) % 16],
                        exv,
                        mask=(it >= 4 * jj) & (it < 4 * jj + 4))
                return carry2

            lax.fori_loop(0, K // 4, group, 0)
            pltpu.sync_copy(exrow_v, ex_hbm.at[pl.ds(base, K)])
            pltpu.sync_copy(exrow16_v, s_spmem.at[idxd_v], add=True)
            return carry

        lax.fori_loop(0, _P1_CHUNKS, chunk, 0)
        plsc.subcore_barrier()

        @pl.when(t < 10)
        def _():
            pltpu.sync_copy(s_spmem.at[pl.ds(t * 5000, 5000)],
                            s_hbm.at[c, pl.ds(t * 5000, 5000)])
    return pl.kernel(
        body,
        out_type=(
            jax.ShapeDtypeStruct((E, HEADS), jnp.float32),
            jax.ShapeDtypeStruct((_NC, N, 16), jnp.float32),
        ),
        mesh=plsc.VectorSubcoreMesh(core_axis_name="c", subcore_axis_name="s"),
        scratch_types=[
            pltpu.VMEM((K,), jnp.int32),
            pltpu.VMEM((K,), jnp.int32),
            pltpu.VMEM((K, 16), jnp.float32),
            pltpu.VMEM((K, 16), jnp.float32),
            pltpu.VMEM((K, HEADS), jnp.float32),
            pltpu.VMEM((K, 16), jnp.float32),
            pltpu.VMEM((16,), jnp.float32),
            pltpu.VMEM_SHARED((N, 16), jnp.float32),
        ],
        compiler_params=pltpu.CompilerParams(
            needs_layout_passes=False, use_tc_tiling_on_sc=False),
    )


_P2_K = 400


def _make_phase2(G, heads_div):
    K = _P2_K
    G2 = G // _NC

    def body(hp_hbm, src_hbm, dst_hbm, ex_hbm, zeros32_hbm, out_hbm,
             idxs_v, idxd_v, hidx_v, exbuf_v, hbuf_v, out_sp):
        c = lax.axis_index("c")
        t = lax.axis_index("s")
        it = _iota16()
        per_tec = E // _NS

        for gi in range(G2):
            g = c * G2 + gi
            hh = g // heads_div

            pltpu.sync_copy(zeros32_hbm.at[pl.ds(t * 3125, 3125)],
                            out_sp.at[pl.ds(t * 3125, 3125)])
            plsc.subcore_barrier()

            def chunk(i, carry):
                base = t * per_tec + i * K
                pltpu.sync_copy(src_hbm.at[pl.ds(base, K)], idxs_v)
                pltpu.sync_copy(dst_hbm.at[pl.ds(base, K)], idxd_v)
                pltpu.sync_copy(ex_hbm.at[pl.ds(base, K)], exbuf_v)

                def mkidx(q, carry2):
                    hidx_v[pl.ds(16 * q, 16)] = (
                        idxs_v[pl.ds(16 * q, 16)] * G + g)
                    return carry2

                lax.fori_loop(0, K // 16, mkidx, 0)
                pltpu.sync_copy(hp_hbm.at[hidx_v], hbuf_v)

                def wgt(q, carry2):
                    zi = jnp.zeros((16,), jnp.int32)
                    for r in range(16):
                        j = 16 * q + r
                        sp = plsc.load_gather(exbuf_v, [zi + j, zi + hh])
                        hbuf_v[j, pl.ds(0, 16)] = hbuf_v[j, pl.ds(0, 16)] * sp
                        hbuf_v[j, pl.ds(16, 16)] = (
                            hbuf_v[j, pl.ds(16, 16)] * sp)
                    return carry2

                lax.fori_loop(0, K // 16, wgt, 0)
                pltpu.sync_copy(hbuf_v, out_sp.at[idxd_v], add=True)
                return carry

            lax.fori_loop(0, E // _NS // K, chunk, 0)
            plsc.subcore_barrier()
            pltpu.sync_copy(out_sp.at[pl.ds(t * 3125, 3125)],
                            out_hbm.at[g, pl.ds(t * 3125, 3125)])
            plsc.subcore_barrier()

    return pl.kernel(
        body,
        out_type=jax.ShapeDtypeStruct((G, N, 32), jnp.float32),
        mesh=plsc.VectorSubcoreMesh(core_axis_name="c", subcore_axis_name="s"),
        scratch_types=[
            pltpu.VMEM((K,), jnp.int32),
            pltpu.VMEM((K,), jnp.int32),
            pltpu.VMEM((K,), jnp.int32),
            pltpu.VMEM((K, HEADS), jnp.float32),
            pltpu.VMEM((K, 32), jnp.float32),
            pltpu.VMEM_SHARED((N, 32), jnp.float32),
        ],
        compiler_params=pltpu.CompilerParams(
            needs_layout_passes=False, use_tc_tiling_on_sc=False),
    )


_phase1 = _make_phase1()
_phase2_4 = _make_phase2(4, 1)
_phase2_8 = _make_phase2(8, 2)


def _gat_conv_sc(h_in, src, dst, n, W, a_src, a_dst, bias,
                 zeros16, zeros32):
    heads = a_src.shape[0]
    C = W.shape[1] // heads
    D = W.shape[1]
    G = D // 32
    hp = h_in @ W
    hph = hp.reshape(n, heads, C)
    al_s = jnp.sum(hph * a_src[None], axis=-1)
    al_d = jnp.sum(hph * a_dst[None], axis=-1)
    als16 = jnp.pad(al_s, ((0, 0), (0, 12)))
    ald16 = jnp.pad(al_d, ((0, 0), (0, 12)))

    ex, s_part = _phase1(als16, ald16, src, dst, zeros16)
    phase2 = _phase2_4 if G == 4 else _phase2_8
    out_g = phase2(hp.reshape(n * G, 32), src, dst, ex, zeros32)

    e_self = al_s + al_d
    ex_self = jnp.exp(jnp.where(e_self > 0, e_self, 0.2 * e_self))
    s_tot = s_part[0, :, :HEADS] + s_part[1, :, :HEADS] + ex_self
    outu = out_g.transpose(1, 0, 2).reshape(n, D).reshape(n, heads, C)
    outu = outu + ex_self[:, :, None] * hph
    return (outu / s_tot[:, :, None]).reshape(n, D) + bias


def _gat_block_sc(h, src, dst, n, p, zeros16, zeros32):
    res = h @ p['rw'] + p['rb'] if 'rw' in p else h
    hh = _gat_conv_sc(h, src, dst, n, p['W'], p['as'], p['ad'], p['b'],
                      zeros16, zeros32)
    hh = _layer_norm(hh, p['ng'], p['nb'])
    return _gelu(hh + res)


def kernel(x, u, params, edge_index, batch):
    n = x.shape[0]
    b = u.shape[0]
    src = edge_index[0]
    dst = edge_index[1]
    p = params
    zeros16 = jnp.zeros((n, 16), jnp.float32)
    zeros32 = jnp.zeros((n, 32), jnp.float32)

    h = _gelu(_layer_norm(x @ p['ne_w'] + p['ne_b'], p['ne_g'], p['ne_be']))
    for name in ('g1', 'g2', 'g3', 'g4'):
        h = _gat_block_sc(h, src, dst, n, p[name], zeros16, zeros32)

    counts = jax.ops.segment_sum(jnp.ones((n,), jnp.float32), batch,
                                 num_segments=b)
    gsum = jax.ops.segment_sum(h, batch, num_segments=b)
    gmean = gsum / jnp.maximum(counts, 1.0)[:, None]
    gmax = jax.ops.segment_max(h, batch, num_segments=b)
    gfeat = jnp.concatenate([gmean, gmax, gsum / 10.0], axis=1)

    g = _gelu(_layer_norm(u @ p['ge_w1'] + p['ge_b1'], p['ge_g1'],
                          p['ge_be1']))
    g = _gelu(_layer_norm(g @ p['ge_w2'] + p['ge_b2'], p['ge_g2'],
                          p['ge_be2']))
    c = jnp.concatenate([gfeat, g], axis=1)
    c = _gelu(_layer_norm(c @ p['fu_w1'] + p['fu_b1'], p['fu_g1'],
                          p['fu_be1']))
    c = _gelu(_layer_norm(c @ p['fu_w2'] + p['fu_b2'], p['fu_g2'],
                          p['fu_be2']))
    c = _gelu(c @ p['cl_w1'] + p['cl_b1'])
    logits = c @ p['cl_w2'] + p['cl_b2']
    return logits[:, 0]


# speedup vs baseline: 33.4525x; 1.0393x over previous
import functools

import jax
import jax.numpy as jnp
from jax import lax
from jax.experimental import pallas as pl
from jax.experimental.pallas import tpu as pltpu
from jax.experimental.pallas import tpu_sc as plsc

N = 50000
E = 800000
HEADS = 4

_NC = 2
_NS = 16


def _iota16():
    return lax.iota(jnp.int32, 16)


def _dot(a, b):
    return jnp.dot(a, b, precision=lax.Precision.HIGHEST)


def _gelu(x):
    return 0.5 * x * (1.0 + lax.erf(x * 0.7071067811865476))


def _layer_norm(x, g, b, eps=1e-5):
    m = jnp.mean(x, axis=-1, keepdims=True)
    v = jnp.var(x, axis=-1, keepdims=True)
    return (x - m) / jnp.sqrt(v + eps) * g + b


_P1_K = 1000
_P1_CHUNKS = E // (_NC * _NS * _P1_K)


def _make_phase1():
    K = _P1_K

    def body(als_hbm, ald_hbm, src_hbm, dst_hbm, zeros16_hbm, ex_hbm, s_hbm,
             idxs_v, idxd_v, sbuf_v, dbuf_v, exrow_v, exrow16_v, merge_v,
             s_spmem):
        c = lax.axis_index("c")
        t = lax.axis_index("s")
        it = _iota16()

        @pl.when(t < 10)
        def _():
            pltpu.sync_copy(zeros16_hbm.at[pl.ds(t * 5000, 5000)],
                            s_spmem.at[pl.ds(t * 5000, 5000)])

        plsc.subcore_barrier()

        per_tec = E // (_NC * _NS)

        def chunk(i, carry):
            base = (c * _NS + t) * per_tec + i * K
            pltpu.sync_copy(src_hbm.at[pl.ds(base, K)], idxs_v)
            pltpu.sync_copy(dst_hbm.at[pl.ds(base, K)], idxd_v)
            pltpu.sync_copy(als_hbm.at[idxs_v], sbuf_v)
            pltpu.sync_copy(ald_hbm.at[idxd_v], dbuf_v)

            def group(q, carry2):
                for jj in range(4):
                    j = 4 * q + jj
                    e = sbuf_v[j] + dbuf_v[j]
                    plsc.store_scatter(merge_v, [(it % 4) + 4 * jj], e,
                                       mask=it < 4)
                m = merge_v[...]
                m = jnp.where(m > 0, m, 0.2 * m)
                exv = jnp.exp(m)
                plsc.store_scatter(exrow_v, [4 * q + it // 4, it % 4], exv)
                for jj in range(4):
                    plsc.store_scatter(
                        exrow16_v,
                        [jnp.full((16,), 4 * q + jj, jnp.int32),
                         (it - 4 * jj) % 16],
                        exv,
                        mask=(it >= 4 * jj) & (it < 4 * jj + 4))
                return carry2

            lax.fori_loop(0, K // 4, group, 0)
            pltpu.sync_copy(exrow_v, ex_hbm.at[pl.ds(base, K)])
            pltpu.sync_copy(exrow16_v, s_spmem.at[idxd_v], add=True)
            return carry

        lax.fori_loop(0, _P1_CHUNKS, chunk, 0)
        plsc.subcore_barrier()

        @pl.when(t < 10)
        def _():
            pltpu.sync_copy(s_spmem.at[pl.ds(t * 5000, 5000)],
                            s_hbm.at[c, pl.ds(t * 5000, 5000)])
    return pl.kernel(
        body,
        out_type=(
            jax.ShapeDtypeStruct((E, HEADS), jnp.float32),
            jax.ShapeDtypeStruct((_NC, N, 16), jnp.float32),
        ),
        mesh=plsc.VectorSubcoreMesh(core_axis_name="c", subcore_axis_name="s"),
        scratch_types=[
            pltpu.VMEM((K,), jnp.int32),
            pltpu.VMEM((K,), jnp.int32),
            pltpu.VMEM((K, 16), jnp.float32),
            pltpu.VMEM((K, 16), jnp.float32),
            pltpu.VMEM((K, HEADS), jnp.float32),
            pltpu.VMEM((K, 16), jnp.float32),
            pltpu.VMEM((16,), jnp.float32),
            pltpu.VMEM_SHARED((N, 16), jnp.float32),
        ],
        compiler_params=pltpu.CompilerParams(
            needs_layout_passes=False, use_tc_tiling_on_sc=False),
    )


_P2_K = 400


def _make_phase2(G, heads_div):
    K = _P2_K
    G2 = G // _NC

    def body(hp_hbm, src_hbm, dst_hbm, ex_hbm, zeros32_hbm, out_hbm,
             idxs_v, idxd_v, hidx_v, exbuf_v, hbuf_v, out_sp):
        c = lax.axis_index("c")
        t = lax.axis_index("s")
        it = _iota16()
        per_tec = E // _NS

        for gi in range(G2):
            g = c * G2 + gi
            hh = g // heads_div

            pltpu.sync_copy(zeros32_hbm.at[pl.ds(t * 3125, 3125)],
                            out_sp.at[pl.ds(t * 3125, 3125)])
            plsc.subcore_barrier()

            def chunk(i, carry):
                base = t * per_tec + i * K
                pltpu.sync_copy(src_hbm.at[pl.ds(base, K)], idxs_v)
                pltpu.sync_copy(dst_hbm.at[pl.ds(base, K)], idxd_v)
                pltpu.sync_copy(ex_hbm.at[pl.ds(base, K)], exbuf_v)

                def mkidx(q, carry2):
                    hidx_v[pl.ds(16 * q, 16)] = (
                        idxs_v[pl.ds(16 * q, 16)] + g * N)
                    return carry2

                lax.fori_loop(0, K // 16, mkidx, 0)
                pltpu.sync_copy(hp_hbm.at[hidx_v], hbuf_v)

                def wgt(q, carry2):
                    zi = jnp.zeros((16,), jnp.int32)
                    for r in range(16):
                        j = 16 * q + r
                        sp = plsc.load_gather(exbuf_v, [zi + j, zi + hh])
                        hbuf_v[j, pl.ds(0, 16)] = hbuf_v[j, pl.ds(0, 16)] * sp
                        hbuf_v[j, pl.ds(16, 16)] = (
                            hbuf_v[j, pl.ds(16, 16)] * sp)
                    return carry2

                lax.fori_loop(0, K // 16, wgt, 0)
                pltpu.sync_copy(hbuf_v, out_sp.at[idxd_v], add=True)
                return carry

            lax.fori_loop(0, E // _NS // K, chunk, 0)
            plsc.subcore_barrier()
            pltpu.sync_copy(out_sp.at[pl.ds(t * 3125, 3125)],
                            out_hbm.at[g, pl.ds(t * 3125, 3125)])
            plsc.subcore_barrier()

    return pl.kernel(
        body,
        out_type=jax.ShapeDtypeStruct((G, N, 32), jnp.float32),
        mesh=plsc.VectorSubcoreMesh(core_axis_name="c", subcore_axis_name="s"),
        scratch_types=[
            pltpu.VMEM((K,), jnp.int32),
            pltpu.VMEM((K,), jnp.int32),
            pltpu.VMEM((K,), jnp.int32),
            pltpu.VMEM((K, HEADS), jnp.float32),
            pltpu.VMEM((K, 32), jnp.float32),
            pltpu.VMEM_SHARED((N, 32), jnp.float32),
        ],
        compiler_params=pltpu.CompilerParams(
            needs_layout_passes=False, use_tc_tiling_on_sc=False),
    )


_phase1 = _make_phase1()
_phase2_4 = _make_phase2(4, 1)
_phase2_8 = _make_phase2(8, 2)


_RB = 1000
_GRID = N // _RB


def _full_spec(shape):
    nd = len(shape)
    return pl.BlockSpec(shape, lambda i: (0,) * nd)


def _row_spec(cols):
    return pl.BlockSpec((_RB, cols), lambda i: (i, 0))


def _enc_kernel(x_ref, w_ref, b_ref, g_ref, be_ref, out_ref):
    h = _dot(x_ref[...], w_ref[...]) + b_ref[...]
    out_ref[...] = _gelu(_layer_norm(h, g_ref[...], be_ref[...]))


def _make_enc(din, dout):
    return pl.pallas_call(
        _enc_kernel,
        grid=(_GRID,),
        in_specs=[_row_spec(din), _full_spec((din, dout)),
                  _full_spec((1, dout)), _full_spec((1, dout)),
                  _full_spec((1, dout))],
        out_specs=_row_spec(dout),
        out_shape=jax.ShapeDtypeStruct((N, dout), jnp.float32),
    )


def _make_pre(din, dout, with_res):
    G = dout // 32

    def body(h_ref, w_ref, as_ref, ad_ref, *rest):
        if with_res:
            rw_ref, rb_ref, hpg_ref, als_ref, ald_ref, res_ref = rest
        else:
            hpg_ref, als_ref, ald_ref = rest
        hb = h_ref[...]
        hp = _dot(hb, w_ref[...])
        als_ref[...] = _dot(hp, as_ref[...])
        ald_ref[...] = _dot(hp, ad_ref[...])
        for g in range(G):
            hpg_ref[g] = hp[:, 32 * g:32 * (g + 1)]
        if with_res:
            res_ref[...] = _dot(hb, rw_ref[...]) + rb_ref[...]

    in_specs = [_row_spec(din), _full_spec((din, dout)),
                _full_spec((dout, 16)), _full_spec((dout, 16))]
    out_shapes = [jax.ShapeDtypeStruct((G, N, 32), jnp.float32),
                  jax.ShapeDtypeStruct((N, 16), jnp.float32),
                  jax.ShapeDtypeStruct((N, 16), jnp.float32)]
    out_specs = [pl.BlockSpec((G, _RB, 32), lambda i: (0, i, 0)),
                 _row_spec(16), _row_spec(16)]
    if with_res:
        in_specs += [_full_spec((din, dout)), _full_spec((1, dout))]
        out_shapes.append(jax.ShapeDtypeStruct((N, dout), jnp.float32))
        out_specs.append(_row_spec(dout))
    return pl.pallas_call(
        body,
        grid=(_GRID,),
        in_specs=in_specs,
        out_specs=out_specs,
        out_shape=out_shapes,
    )


def _make_post(dout):
    G = dout // 32
    C = dout // HEADS

    def body(outg_ref, s_ref, als_ref, ald_ref, hpg_ref, res_ref,
             bias_ref, ng_ref, nb_ref, out_ref):
        un = jnp.concatenate([outg_ref[g] for g in range(G)], axis=1)
        hp = jnp.concatenate([hpg_ref[g] for g in range(G)], axis=1)
        es = als_ref[...][:, :HEADS] + ald_ref[...][:, :HEADS]
        es = jnp.where(es > 0, es, 0.2 * es)
        exs = jnp.exp(es)
        stot = s_ref[0][:, :HEADS] + s_ref[1][:, :HEADS] + exs
        parts = []
        for h in range(HEADS):
            lo = h * C
            num = (un[:, lo:lo + C]
                   + exs[:, h:h + 1] * hp[:, lo:lo + C])
            parts.append(num / stot[:, h:h + 1])
        gat = jnp.concatenate(parts, axis=1) + bias_ref[...]
        gat = _layer_norm(gat, ng_ref[...], nb_ref[...])
        out_ref[...] = _gelu(gat + res_ref[...])

    return pl.pallas_call(
        body,
        grid=(_GRID,),
        in_specs=[pl.BlockSpec((G, _RB, 32), lambda i: (0, i, 0)),
                  pl.BlockSpec((2, _RB, 16), lambda i: (0, i, 0)),
                  _row_spec(16), _row_spec(16),
                  pl.BlockSpec((G, _RB, 32), lambda i: (0, i, 0)),
                  _row_spec(dout),
                  _full_spec((1, dout)), _full_spec((1, dout)),
                  _full_spec((1, dout))],
        out_specs=_row_spec(dout),
        out_shape=jax.ShapeDtypeStruct((N, dout), jnp.float32),
    )


def _pool_kernel(h_ref, b_ref, sum_ref, max_ref, cnt_ref):
    i = pl.program_id(0)
    hb = h_ref[...]
    seg = lax.broadcasted_iota(jnp.int32, (_RB, 32), 1)
    mask = b_ref[...] == seg
    maskf = mask.astype(jnp.float32)
    dims = (((0,), (0,)), ((), ()))
    psum = lax.dot_general(maskf, hb, dims, precision=lax.Precision.HIGHEST)
    pcnt = lax.dot_general(maskf, jnp.ones_like(hb), dims, precision=lax.Precision.HIGHEST)
    pms = []
    for s in range(32):
        hv = jnp.where(mask[:, s:s + 1], hb, -jnp.inf)
        pms.append(jnp.max(hv, axis=0, keepdims=True))
    pmax = jnp.concatenate(pms, axis=0)

    @pl.when(i == 0)
    def _():
        sum_ref[...] = psum
        max_ref[...] = pmax
        cnt_ref[...] = pcnt

    @pl.when(i != 0)
    def _():
        sum_ref[...] += psum
        max_ref[...] = jnp.maximum(max_ref[...], pmax)
        cnt_ref[...] += pcnt


_pool = pl.pallas_call(
    _pool_kernel,
    grid=(_GRID,),
    in_specs=[_row_spec(128), _row_spec(1)],
    out_specs=[_full_spec((32, 128))] * 3,
    out_shape=[jax.ShapeDtypeStruct((32, 128), jnp.float32)] * 3,
)


def _mlp_kernel(gsum_ref, gmax_ref, cnt_ref, u_ref,
                gw1_ref, gb1_ref, gg1_ref, gbe1_ref,
                gw2_ref, gb2_ref, gg2_ref, gbe2_ref,
                fw1_ref, fb1_ref, fg1_ref, fbe1_ref,
                fw2_ref, fb2_ref, fg2_ref, fbe2_ref,
                cw1_ref, cb1_ref, cw2_ref, cb2_ref, out_ref):
    gsum = gsum_ref[...]
    gmean = gsum / jnp.maximum(cnt_ref[...], 1.0)
    gfeat = jnp.concatenate([gmean, gmax_ref[...], gsum / 10.0], axis=1)
    u = u_ref[...]
    g = _gelu(_layer_norm(_dot(u, gw1_ref[...]) + gb1_ref[...], gg1_ref[...],
                          gbe1_ref[...]))
    g = _gelu(_layer_norm(_dot(g, gw2_ref[...]) + gb2_ref[...], gg2_ref[...],
                          gbe2_ref[...]))
    c = jnp.concatenate([gfeat, g], axis=1)
    c = _gelu(_layer_norm(_dot(c, fw1_ref[...]) + fb1_ref[...], fg1_ref[...],
                          fbe1_ref[...]))
    c = _gelu(_layer_norm(_dot(c, fw2_ref[...]) + fb2_ref[...], fg2_ref[...],
                          fbe2_ref[...]))
    c = _gelu(_dot(c, cw1_ref[...]) + cb1_ref[...])
    out_ref[...] = _dot(c, cw2_ref[...]) + cb2_ref[...]


_enc = _make_enc(16, 64)
_pre_128 = _make_pre(64, 128, True)
_pre_256r = _make_pre(128, 256, True)
_pre_256 = _make_pre(256, 256, False)
_pre_128b = _make_pre(256, 128, True)
_post_128 = _make_post(128)
_post_256 = _make_post(256)


def _attn_mat(a, dout):
    heads, C = a.shape
    eye = jnp.eye(heads, 16, dtype=jnp.float32)
    return (a[:, :, None] * eye[:, None, :]).reshape(dout, 16)


def _gat_block_sc(h, src, dst, n, p, pre, post, zeros16, zeros32):
    D = p['W'].shape[1]
    G = D // 32
    As16 = _attn_mat(p['as'], D)
    Ad16 = _attn_mat(p['ad'], D)
    if 'rw' in p:
        hp_g, als16, ald16, res = pre(h, p['W'], As16, Ad16, p['rw'],
                                      p['rb'][None])
    else:
        hp_g, als16, ald16 = pre(h, p['W'], As16, Ad16)
        res = h
    ex, s_part = _phase1(als16, ald16, src, dst, zeros16)
    phase2 = _phase2_4 if G == 4 else _phase2_8
    out_g = phase2(hp_g.reshape(G * n, 32), src, dst, ex, zeros32)
    return post(out_g, s_part, als16, ald16, hp_g, res,
                p['b'][None], p['ng'][None], p['nb'][None])


def kernel(x, u, params, edge_index, batch):
    n = x.shape[0]
    src = edge_index[0]
    dst = edge_index[1]
    p = params
    zeros16 = jnp.zeros((n, 16), jnp.float32)
    zeros32 = jnp.zeros((n, 32), jnp.float32)

    h = _enc(x, p['ne_w'], p['ne_b'][None], p['ne_g'][None], p['ne_be'][None])
    for name, pre, post in (('g1', _pre_128, _post_128),
                            ('g2', _pre_256r, _post_256),
                            ('g3', _pre_256, _post_256),
                            ('g4', _pre_128b, _post_128)):
        h = _gat_block_sc(h, src, dst, n, p[name], pre, post,
                          zeros16, zeros32)

    gsum, gmax, cnt = _pool(h, batch[:, None].astype(jnp.int32))

    out = pl.pallas_call(
        _mlp_kernel,
        out_shape=jax.ShapeDtypeStruct((32, 1), jnp.float32),
    )(gsum, gmax, cnt, u,
      p['ge_w1'], p['ge_b1'], p['ge_g1'], p['ge_be1'],
      p['ge_w2'], p['ge_b2'], p['ge_g2'], p['ge_be2'],
      p['fu_w1'], p['fu_b1'], p['fu_g1'], p['fu_be1'],
      p['fu_w2'], p['fu_b2'], p['fu_g2'], p['fu_be2'],
      p['cl_w1'], p['cl_b1'], p['cl_w2'], p['cl_b2'])
    return out[:, 0]


# speedup vs baseline: 49.7822x; 1.4881x over previous
import functools

import jax
import jax.numpy as jnp
from jax import lax
from jax.experimental import pallas as pl
from jax.experimental.pallas import tpu as pltpu
from jax.experimental.pallas import tpu_sc as plsc

N = 50000
E = 800000
HEADS = 4

_NC = 2
_NS = 16


def _iota16():
    return lax.iota(jnp.int32, 16)


def _dot(a, b):
    return jnp.dot(a, b, precision=lax.Precision.HIGHEST)


def _gelu(x):
    return 0.5 * x * (1.0 + lax.erf(x * 0.7071067811865476))


def _layer_norm(x, g, b, eps=1e-5):
    m = jnp.mean(x, axis=-1, keepdims=True)
    v = jnp.var(x, axis=-1, keepdims=True)
    return (x - m) / jnp.sqrt(v + eps) * g + b


_P1_K = 1000
_P1_CHUNKS = E // (_NC * _NS * _P1_K)


def _make_phase1():
    K = _P1_K

    def body(als_hbm, ald_hbm, src_hbm, dst_hbm, zeros16_hbm, ex_hbm, s_hbm,
             idxs_v, idxd_v, sbuf_v, dbuf_v, exrow_v, exrow16_v, merge_v,
             s_spmem):
        c = lax.axis_index("c")
        t = lax.axis_index("s")
        it = _iota16()

        @pl.when(t < 10)
        def _():
            pltpu.sync_copy(zeros16_hbm.at[pl.ds(t * 5000, 5000)],
                            s_spmem.at[pl.ds(t * 5000, 5000)])

        plsc.subcore_barrier()

        per_tec = E // (_NC * _NS)

        def chunk(i, carry):
            base = (c * _NS + t) * per_tec + i * K
            pltpu.sync_copy(src_hbm.at[pl.ds(base, K)], idxs_v)
            pltpu.sync_copy(dst_hbm.at[pl.ds(base, K)], idxd_v)
            pltpu.sync_copy(als_hbm.at[idxs_v], sbuf_v)
            pltpu.sync_copy(ald_hbm.at[idxd_v], dbuf_v)

            def group(q, carry2):
                for jj in range(4):
                    j = 4 * q + jj
                    e = sbuf_v[j] + dbuf_v[j]
                    plsc.store_scatter(merge_v, [(it % 4) + 4 * jj], e,
                                       mask=it < 4)
                m = merge_v[...]
                m = jnp.where(m > 0, m, 0.2 * m)
                exv = jnp.exp(m)
                plsc.store_scatter(exrow_v, [it % 4, 4 * q + it // 4], exv)
                for jj in range(4):
                    plsc.store_scatter(
                        exrow16_v,
                        [jnp.full((16,), 4 * q + jj, jnp.int32),
                         (it - 4 * jj) % 16],
                        exv,
                        mask=(it >= 4 * jj) & (it < 4 * jj + 4))
                return carry2

            lax.fori_loop(0, K // 4, group, 0)
            pltpu.sync_copy(exrow_v, ex_hbm.at[:, pl.ds(base, K)])
            pltpu.sync_copy(exrow16_v, s_spmem.at[idxd_v], add=True)
            return carry

        lax.fori_loop(0, _P1_CHUNKS, chunk, 0)
        plsc.subcore_barrier()

        @pl.when(t < 10)
        def _():
            pltpu.sync_copy(s_spmem.at[pl.ds(t * 5000, 5000)],
                            s_hbm.at[c, pl.ds(t * 5000, 5000)])
    return pl.kernel(
        body,
        out_type=(
            jax.ShapeDtypeStruct((HEADS, E), jnp.float32),
            jax.ShapeDtypeStruct((_NC, N, 16), jnp.float32),
        ),
        mesh=plsc.VectorSubcoreMesh(core_axis_name="c", subcore_axis_name="s"),
        scratch_types=[
            pltpu.VMEM((K,), jnp.int32),
            pltpu.VMEM((K,), jnp.int32),
            pltpu.VMEM((K, 16), jnp.float32),
            pltpu.VMEM((K, 16), jnp.float32),
            pltpu.VMEM((HEADS, K), jnp.float32),
            pltpu.VMEM((K, 16), jnp.float32),
            pltpu.VMEM((16,), jnp.float32),
            pltpu.VMEM_SHARED((N, 16), jnp.float32),
        ],
        compiler_params=pltpu.CompilerParams(
            needs_layout_passes=False, use_tc_tiling_on_sc=False),
    )


_P2_K = 400


def _make_phase2(G, heads_div):
    K = _P2_K
    G2 = G // _NC

    NCH = E // _NS // K

    def body(hp_hbm, src_hbm, dst_hbm, ex_hbm, zeros32_hbm, out_hbm,
             idxs_v, idxd_v, hidx_v, exbuf_v, hbuf_v, out_sp, sem_l, sem_g):
        c = lax.axis_index("c")
        t = lax.axis_index("s")
        per_tec = E // _NS

        def loads_desc(i, hh):
            base = t * per_tec + i * K
            s = i % 3
            return (pltpu.make_async_copy(src_hbm.at[pl.ds(base, K)],
                                          idxs_v, sem_l),
                    pltpu.make_async_copy(dst_hbm.at[pl.ds(base, K)],
                                          idxd_v.at[s], sem_l),
                    pltpu.make_async_copy(ex_hbm.at[hh, pl.ds(base, K)],
                                          exbuf_v.at[s], sem_l))

        def gather_desc(slot):
            return pltpu.make_async_copy(hp_hbm.at[hidx_v],
                                         hbuf_v.at[slot], sem_g)

        def mkidx(g):
            def step(q, carry2):
                hidx_v[pl.ds(16 * q, 16)] = (
                    idxs_v[pl.ds(16 * q, 16)] + g * N)
                return carry2
            lax.fori_loop(0, K // 16, step, 0)

        for gi in range(G2):
            g = c * G2 + gi
            hh = g // heads_div

            pltpu.sync_copy(zeros32_hbm.at[pl.ds(t * 3125, 3125)],
                            out_sp.at[pl.ds(t * 3125, 3125)])
            plsc.subcore_barrier()

            for d in loads_desc(0, hh):
                d.start()
            for d in loads_desc(0, hh):
                d.wait()
            mkidx(g)
            gather_desc(0).start()
            for d in loads_desc(1, hh):
                d.start()

            def chunk(i, carry):
                b = i % 2
                nb = 1 - b
                s = i % 3
                gather_desc(b).wait()

                @pl.when(i + 1 < NCH)
                def _():
                    for d in loads_desc(i + 1, hh):
                        d.wait()
                    mkidx(g)
                    gather_desc(nb).start()

                @pl.when(i + 2 < NCH)
                def _():
                    for d in loads_desc(i + 2, hh):
                        d.start()

                def wgt(q, carry2):
                    zi = jnp.zeros((16,), jnp.int32)
                    for r in range(16):
                        j = 16 * q + r
                        sp = plsc.load_gather(exbuf_v, [zi + s, zi + j])
                        hbuf_v[b, j, pl.ds(0, 16)] = (
                            hbuf_v[b, j, pl.ds(0, 16)] * sp)
                        hbuf_v[b, j, pl.ds(16, 16)] = (
                            hbuf_v[b, j, pl.ds(16, 16)] * sp)
                    return carry2

                lax.fori_loop(0, K // 16, wgt, 0)
                pltpu.sync_copy(hbuf_v.at[b], out_sp.at[idxd_v.at[s]],
                                add=True)
                return carry

            lax.fori_loop(0, NCH, chunk, 0)
            plsc.subcore_barrier()
            pltpu.sync_copy(out_sp.at[pl.ds(t * 3125, 3125)],
                            out_hbm.at[g, pl.ds(t * 3125, 3125)])
            plsc.subcore_barrier()

    return pl.kernel(
        body,
        out_type=jax.ShapeDtypeStruct((G, N, 32), jnp.float32),
        mesh=plsc.VectorSubcoreMesh(core_axis_name="c", subcore_axis_name="s"),
        scratch_types=[
            pltpu.VMEM((K,), jnp.int32),
            pltpu.VMEM((3, K), jnp.int32),
            pltpu.VMEM((K,), jnp.int32),
            pltpu.VMEM((3, K), jnp.float32),
            pltpu.VMEM((2, K, 32), jnp.float32),
            pltpu.VMEM_SHARED((N, 32), jnp.float32),
            pltpu.SemaphoreType.DMA,
            pltpu.SemaphoreType.DMA,
        ],
        compiler_params=pltpu.CompilerParams(
            needs_layout_passes=False, use_tc_tiling_on_sc=False),
    )


_phase1 = _make_phase1()
_phase2_4 = _make_phase2(4, 1)
_phase2_8 = _make_phase2(8, 2)


_RB = 1000
_GRID = N // _RB


def _full_spec(shape):
    nd = len(shape)
    return pl.BlockSpec(shape, lambda i: (0,) * nd)


def _row_spec(cols):
    return pl.BlockSpec((_RB, cols), lambda i: (i, 0))


def _enc_kernel(x_ref, w_ref, b_ref, g_ref, be_ref, out_ref):
    h = _dot(x_ref[...], w_ref[...]) + b_ref[...]
    out_ref[...] = _gelu(_layer_norm(h, g_ref[...], be_ref[...]))


def _make_enc(din, dout):
    return pl.pallas_call(
        _enc_kernel,
        grid=(_GRID,),
        in_specs=[_row_spec(din), _full_spec((din, dout)),
                  _full_spec((1, dout)), _full_spec((1, dout)),
                  _full_spec((1, dout))],
        out_specs=_row_spec(dout),
        out_shape=jax.ShapeDtypeStruct((N, dout), jnp.float32),
    )


def _make_pre(din, dout, with_res):
    G = dout // 32

    def body(h_ref, w_ref, as_ref, ad_ref, *rest):
        if with_res:
            rw_ref, rb_ref, hpg_ref, als_ref, ald_ref, res_ref = rest
        else:
            hpg_ref, als_ref, ald_ref = rest
        hb = h_ref[...]
        hp = _dot(hb, w_ref[...])
        als_ref[...] = _dot(hp, as_ref[...])
        ald_ref[...] = _dot(hp, ad_ref[...])
        for g in range(G):
            hpg_ref[g] = hp[:, 32 * g:32 * (g + 1)]
        if with_res:
            res_ref[...] = _dot(hb, rw_ref[...]) + rb_ref[...]

    in_specs = [_row_spec(din), _full_spec((din, dout)),
                _full_spec((dout, 16)), _full_spec((dout, 16))]
    out_shapes = [jax.ShapeDtypeStruct((G, N, 32), jnp.float32),
                  jax.ShapeDtypeStruct((N, 16), jnp.float32),
                  jax.ShapeDtypeStruct((N, 16), jnp.float32)]
    out_specs = [pl.BlockSpec((G, _RB, 32), lambda i: (0, i, 0)),
                 _row_spec(16), _row_spec(16)]
    if with_res:
        in_specs += [_full_spec((din, dout)), _full_spec((1, dout))]
        out_shapes.append(jax.ShapeDtypeStruct((N, dout), jnp.float32))
        out_specs.append(_row_spec(dout))
    return pl.pallas_call(
        body,
        grid=(_GRID,),
        in_specs=in_specs,
        out_specs=out_specs,
        out_shape=out_shapes,
    )


def _make_post(dout):
    G = dout // 32
    C = dout // HEADS

    def body(outg_ref, s_ref, als_ref, ald_ref, hpg_ref, res_ref,
             bias_ref, ng_ref, nb_ref, out_ref):
        un = jnp.concatenate([outg_ref[g] for g in range(G)], axis=1)
        hp = jnp.concatenate([hpg_ref[g] for g in range(G)], axis=1)
        es = als_ref[...][:, :HEADS] + ald_ref[...][:, :HEADS]
        es = jnp.where(es > 0, es, 0.2 * es)
        exs = jnp.exp(es)
        stot = s_ref[0][:, :HEADS] + s_ref[1][:, :HEADS] + exs
        parts = []
        for h in range(HEADS):
            lo = h * C
            num = (un[:, lo:lo + C]
                   + exs[:, h:h + 1] * hp[:, lo:lo + C])
            parts.append(num / stot[:, h:h + 1])
        gat = jnp.concatenate(parts, axis=1) + bias_ref[...]
        gat = _layer_norm(gat, ng_ref[...], nb_ref[...])
        out_ref[...] = _gelu(gat + res_ref[...])

    return pl.pallas_call(
        body,
        grid=(_GRID,),
        in_specs=[pl.BlockSpec((G, _RB, 32), lambda i: (0, i, 0)),
                  pl.BlockSpec((2, _RB, 16), lambda i: (0, i, 0)),
                  _row_spec(16), _row_spec(16),
                  pl.BlockSpec((G, _RB, 32), lambda i: (0, i, 0)),
                  _row_spec(dout),
                  _full_spec((1, dout)), _full_spec((1, dout)),
                  _full_spec((1, dout))],
        out_specs=_row_spec(dout),
        out_shape=jax.ShapeDtypeStruct((N, dout), jnp.float32),
    )


def _pool_kernel(h_ref, b_ref, sum_ref, max_ref, cnt_ref):
    i = pl.program_id(0)
    hb = h_ref[...]
    seg = lax.broadcasted_iota(jnp.int32, (_RB, 32), 1)
    mask = b_ref[...] == seg
    maskf = mask.astype(jnp.float32)
    dims = (((0,), (0,)), ((), ()))
    psum = lax.dot_general(maskf, hb, dims, precision=lax.Precision.HIGHEST)
    pcnt = lax.dot_general(maskf, jnp.ones_like(hb), dims, precision=lax.Precision.HIGHEST)
    pms = []
    for s in range(32):
        hv = jnp.where(mask[:, s:s + 1], hb, -jnp.inf)
        pms.append(jnp.max(hv, axis=0, keepdims=True))
    pmax = jnp.concatenate(pms, axis=0)

    @pl.when(i == 0)
    def _():
        sum_ref[...] = psum
        max_ref[...] = pmax
        cnt_ref[...] = pcnt

    @pl.when(i != 0)
    def _():
        sum_ref[...] += psum
        max_ref[...] = jnp.maximum(max_ref[...], pmax)
        cnt_ref[...] += pcnt


_pool = pl.pallas_call(
    _pool_kernel,
    grid=(_GRID,),
    in_specs=[_row_spec(128), _row_spec(1)],
    out_specs=[_full_spec((32, 128))] * 3,
    out_shape=[jax.ShapeDtypeStruct((32, 128), jnp.float32)] * 3,
)


def _mlp_kernel(gsum_ref, gmax_ref, cnt_ref, u_ref,
                gw1_ref, gb1_ref, gg1_ref, gbe1_ref,
                gw2_ref, gb2_ref, gg2_ref, gbe2_ref,
                fw1_ref, fb1_ref, fg1_ref, fbe1_ref,
                fw2_ref, fb2_ref, fg2_ref, fbe2_ref,
                cw1_ref, cb1_ref, cw2_ref, cb2_ref, out_ref):
    gsum = gsum_ref[...]
    gmean = gsum / jnp.maximum(cnt_ref[...], 1.0)
    gfeat = jnp.concatenate([gmean, gmax_ref[...], gsum / 10.0], axis=1)
    u = u_ref[...]
    g = _gelu(_layer_norm(_dot(u, gw1_ref[...]) + gb1_ref[...], gg1_ref[...],
                          gbe1_ref[...]))
    g = _gelu(_layer_norm(_dot(g, gw2_ref[...]) + gb2_ref[...], gg2_ref[...],
                          gbe2_ref[...]))
    c = jnp.concatenate([gfeat, g], axis=1)
    c = _gelu(_layer_norm(_dot(c, fw1_ref[...]) + fb1_ref[...], fg1_ref[...],
                          fbe1_ref[...]))
    c = _gelu(_layer_norm(_dot(c, fw2_ref[...]) + fb2_ref[...], fg2_ref[...],
                          fbe2_ref[...]))
    c = _gelu(_dot(c, cw1_ref[...]) + cb1_ref[...])
    out_ref[...] = _dot(c, cw2_ref[...]) + cb2_ref[...]


_enc = _make_enc(16, 64)
_pre_128 = _make_pre(64, 128, True)
_pre_256r = _make_pre(128, 256, True)
_pre_256 = _make_pre(256, 256, False)
_pre_128b = _make_pre(256, 128, True)
_post_128 = _make_post(128)
_post_256 = _make_post(256)


def _attn_mat(a, dout):
    heads, C = a.shape
    eye = jnp.eye(heads, 16, dtype=jnp.float32)
    return (a[:, :, None] * eye[:, None, :]).reshape(dout, 16)


def _gat_block_sc(h, src, dst, n, p, pre, post, zeros16, zeros32):
    D = p['W'].shape[1]
    G = D // 32
    As16 = _attn_mat(p['as'], D)
    Ad16 = _attn_mat(p['ad'], D)
    if 'rw' in p:
        hp_g, als16, ald16, res = pre(h, p['W'], As16, Ad16, p['rw'],
                                      p['rb'][None])
    else:
        hp_g, als16, ald16 = pre(h, p['W'], As16, Ad16)
        res = h
    ex, s_part = _phase1(als16, ald16, src, dst, zeros16)
    phase2 = _phase2_4 if G == 4 else _phase2_8
    out_g = phase2(hp_g.reshape(G * n, 32), src, dst, ex, zeros32)
    return post(out_g, s_part, als16, ald16, hp_g, res,
                p['b'][None], p['ng'][None], p['nb'][None])


def kernel(x, u, params, edge_index, batch):
    n = x.shape[0]
    src = edge_index[0]
    dst = edge_index[1]
    p = params
    zeros16 = jnp.zeros((n, 16), jnp.float32)
    zeros32 = jnp.zeros((n, 32), jnp.float32)

    h = _enc(x, p['ne_w'], p['ne_b'][None], p['ne_g'][None], p['ne_be'][None])
    for name, pre, post in (('g1', _pre_128, _post_128),
                            ('g2', _pre_256r, _post_256),
                            ('g3', _pre_256, _post_256),
                            ('g4', _pre_128b, _post_128)):
        h = _gat_block_sc(h, src, dst, n, p[name], pre, post,
                          zeros16, zeros32)

    gsum, gmax, cnt = _pool(h, batch[:, None].astype(jnp.int32))

    out = pl.pallas_call(
        _mlp_kernel,
        out_shape=jax.ShapeDtypeStruct((32, 1), jnp.float32),
    )(gsum, gmax, cnt, u,
      p['ge_w1'], p['ge_b1'], p['ge_g1'], p['ge_be1'],
      p['ge_w2'], p['ge_b2'], p['ge_g2'], p['ge_be2'],
      p['fu_w1'], p['fu_b1'], p['fu_g1'], p['fu_be1'],
      p['fu_w2'], p['fu_b2'], p['fu_g2'], p['fu_be2'],
      p['cl_w1'], p['cl_b1'], p['cl_w2'], p['cl_b2'])
    return out[:, 0]


# speedup vs baseline: 52.2932x; 1.0504x over previous
import functools

import jax
import jax.numpy as jnp
from jax import lax
from jax.experimental import pallas as pl
from jax.experimental.pallas import tpu as pltpu
from jax.experimental.pallas import tpu_sc as plsc

N = 50000
E = 800000
HEADS = 4

_NC = 2
_NS = 16


def _iota16():
    return lax.iota(jnp.int32, 16)


def _dot(a, b):
    return jnp.dot(a, b, precision=lax.Precision.HIGHEST)


def _gelu(x):
    return 0.5 * x * (1.0 + lax.erf(x * 0.7071067811865476))


def _layer_norm(x, g, b, eps=1e-5):
    m = jnp.mean(x, axis=-1, keepdims=True)
    v = jnp.var(x, axis=-1, keepdims=True)
    return (x - m) / jnp.sqrt(v + eps) * g + b


_P1_K = 1000
_P1_CHUNKS = E // (_NC * _NS * _P1_K)


def _make_phase1():
    K = _P1_K

    def body(als_hbm, ald_hbm, src_hbm, dst_hbm, zeros16_hbm, ex_hbm, s_hbm,
             idxs_v, idxd_v, sbuf_v, dbuf_v, exrow_v, exrow16_v, merge_v,
             s_spmem):
        c = lax.axis_index("c")
        t = lax.axis_index("s")
        it = _iota16()

        @pl.when(t < 10)
        def _():
            pltpu.sync_copy(zeros16_hbm.at[pl.ds(t * 5000, 5000)],
                            s_spmem.at[pl.ds(t * 5000, 5000)])

        plsc.subcore_barrier()

        per_tec = E // (_NC * _NS)

        def chunk(i, carry):
            base = (c * _NS + t) * per_tec + i * K
            pltpu.sync_copy(src_hbm.at[pl.ds(base, K)], idxs_v)
            pltpu.sync_copy(dst_hbm.at[pl.ds(base, K)], idxd_v)
            pltpu.sync_copy(als_hbm.at[idxs_v], sbuf_v)
            pltpu.sync_copy(ald_hbm.at[idxd_v], dbuf_v)

            def group(q, carry2):
                for jj in range(4):
                    j = 4 * q + jj
                    e = sbuf_v[j] + dbuf_v[j]
                    plsc.store_scatter(merge_v, [(it % 4) + 4 * jj], e,
                                       mask=it < 4)
                m = merge_v[...]
                m = jnp.where(m > 0, m, 0.2 * m)
                exv = jnp.exp(m)
                plsc.store_scatter(exrow_v, [it % 4, 4 * q + it // 4], exv)
                for jj in range(4):
                    plsc.store_scatter(
                        exrow16_v,
                        [jnp.full((16,), 4 * q + jj, jnp.int32),
                         (it - 4 * jj) % 16],
                        exv,
                        mask=(it >= 4 * jj) & (it < 4 * jj + 4))
                return carry2

            lax.fori_loop(0, K // 4, group, 0)
            pltpu.sync_copy(exrow_v, ex_hbm.at[:, pl.ds(base, K)])
            pltpu.sync_copy(exrow16_v, s_spmem.at[idxd_v], add=True)
            return carry

        lax.fori_loop(0, _P1_CHUNKS, chunk, 0)
        plsc.subcore_barrier()

        @pl.when(t < 10)
        def _():
            pltpu.sync_copy(s_spmem.at[pl.ds(t * 5000, 5000)],
                            s_hbm.at[c, pl.ds(t * 5000, 5000)])
    return pl.kernel(
        body,
        out_type=(
            jax.ShapeDtypeStruct((HEADS, E), jnp.float32),
            jax.ShapeDtypeStruct((_NC, N, 16), jnp.float32),
        ),
        mesh=plsc.VectorSubcoreMesh(core_axis_name="c", subcore_axis_name="s"),
        scratch_types=[
            pltpu.VMEM((K,), jnp.int32),
            pltpu.VMEM((K,), jnp.int32),
            pltpu.VMEM((K, 16), jnp.float32),
            pltpu.VMEM((K, 16), jnp.float32),
            pltpu.VMEM((HEADS, K), jnp.float32),
            pltpu.VMEM((K, 16), jnp.float32),
            pltpu.VMEM((16,), jnp.float32),
            pltpu.VMEM_SHARED((N, 16), jnp.float32),
        ],
        compiler_params=pltpu.CompilerParams(
            needs_layout_passes=False, use_tc_tiling_on_sc=False),
    )


_P2_K = 400


def _make_phase2(G, heads_div):
    K = _P2_K
    G2 = G // _NC

    NCH = E // _NS // K

    def body(hp_hbm, src_hbm, dst_hbm, ex_hbm, zeros32_hbm, out_hbm,
             idxs_v, idxd_v, hidx_v, exbuf_v, hbuf_v, out_sp, sem_l, sem_g):
        c = lax.axis_index("c")
        t = lax.axis_index("s")
        per_tec = E // _NS

        def loads_desc(i, hh):
            base = t * per_tec + i * K
            s = i % 3
            return (pltpu.make_async_copy(src_hbm.at[pl.ds(base, K)],
                                          idxs_v, sem_l),
                    pltpu.make_async_copy(dst_hbm.at[pl.ds(base, K)],
                                          idxd_v.at[s], sem_l),
                    pltpu.make_async_copy(ex_hbm.at[hh, pl.ds(base, K)],
                                          exbuf_v.at[s], sem_l))

        def gather_desc(slot):
            return pltpu.make_async_copy(hp_hbm.at[hidx_v],
                                         hbuf_v.at[slot], sem_g)

        def mkidx(g):
            def step(q, carry2):
                hidx_v[pl.ds(16 * q, 16)] = (
                    idxs_v[pl.ds(16 * q, 16)] + g * N)
                return carry2
            lax.fori_loop(0, K // 16, step, 0)

        for gi in range(G2):
            g = c * G2 + gi
            hh = g // heads_div

            pltpu.sync_copy(zeros32_hbm.at[pl.ds(t * 3125, 3125)],
                            out_sp.at[pl.ds(t * 3125, 3125)])
            plsc.subcore_barrier()

            for d in loads_desc(0, hh):
                d.start()
            for d in loads_desc(0, hh):
                d.wait()
            mkidx(g)
            gather_desc(0).start()
            for d in loads_desc(1, hh):
                d.start()

            def chunk(i, carry):
                b = i % 2
                nb = 1 - b
                s = i % 3
                gather_desc(b).wait()

                @pl.when(i + 1 < NCH)
                def _():
                    for d in loads_desc(i + 1, hh):
                        d.wait()
                    mkidx(g)
                    gather_desc(nb).start()

                @pl.when(i + 2 < NCH)
                def _():
                    for d in loads_desc(i + 2, hh):
                        d.start()

                def wgt(q, carry2):
                    zi = jnp.zeros((16,), jnp.int32)
                    for r in range(16):
                        j = 16 * q + r
                        sp = plsc.load_gather(exbuf_v, [zi + s, zi + j])
                        hbuf_v[b, j, pl.ds(0, 16)] = (
                            hbuf_v[b, j, pl.ds(0, 16)] * sp)
                        hbuf_v[b, j, pl.ds(16, 16)] = (
                            hbuf_v[b, j, pl.ds(16, 16)] * sp)
                    return carry2

                lax.fori_loop(0, K // 16, wgt, 0)
                pltpu.sync_copy(hbuf_v.at[b], out_sp.at[idxd_v.at[s]],
                                add=True)
                return carry

            lax.fori_loop(0, NCH, chunk, 0)
            plsc.subcore_barrier()
            pltpu.sync_copy(out_sp.at[pl.ds(t * 3125, 3125)],
                            out_hbm.at[g, pl.ds(t * 3125, 3125)])
            plsc.subcore_barrier()

    return pl.kernel(
        body,
        out_type=jax.ShapeDtypeStruct((G, N, 32), jnp.float32),
        mesh=plsc.VectorSubcoreMesh(core_axis_name="c", subcore_axis_name="s"),
        scratch_types=[
            pltpu.VMEM((K,), jnp.int32),
            pltpu.VMEM((3, K), jnp.int32),
            pltpu.VMEM((K,), jnp.int32),
            pltpu.VMEM((3, K), jnp.float32),
            pltpu.VMEM((2, K, 32), jnp.float32),
            pltpu.VMEM_SHARED((N, 32), jnp.float32),
            pltpu.SemaphoreType.DMA,
            pltpu.SemaphoreType.DMA,
        ],
        compiler_params=pltpu.CompilerParams(
            needs_layout_passes=False, use_tc_tiling_on_sc=False),
    )


_phase1 = _make_phase1()
_phase2_4 = _make_phase2(4, 1)
_phase2_8 = _make_phase2(8, 2)


_RB = 1000
_GRID = N // _RB


def _full_spec(shape):
    nd = len(shape)
    return pl.BlockSpec(shape, lambda i: (0,) * nd)


def _row_spec(cols):
    return pl.BlockSpec((_RB, cols), lambda i: (i, 0))


def _enc_kernel(x_ref, w_ref, b_ref, g_ref, be_ref, out_ref):
    h = x_ref[...] @ w_ref[...] + b_ref[...]
    out_ref[...] = _gelu(_layer_norm(h, g_ref[...], be_ref[...]))


def _make_enc(din, dout):
    return pl.pallas_call(
        _enc_kernel,
        grid=(_GRID,),
        in_specs=[_row_spec(din), _full_spec((din, dout)),
                  _full_spec((1, dout)), _full_spec((1, dout)),
                  _full_spec((1, dout))],
        out_specs=_row_spec(dout),
        out_shape=jax.ShapeDtypeStruct((N, dout), jnp.float32),
    )


def _make_pre(din, dout, with_res):
    G = dout // 32

    def body(h_ref, w_ref, as_ref, ad_ref, *rest):
        if with_res:
            rw_ref, rb_ref, hpg_ref, als_ref, ald_ref, res_ref = rest
        else:
            hpg_ref, als_ref, ald_ref = rest
        hb = h_ref[...]
        hp = hb @ w_ref[...]
        als_ref[...] = _dot(hp, as_ref[...])
        ald_ref[...] = _dot(hp, ad_ref[...])
        for g in range(G):
            hpg_ref[g] = hp[:, 32 * g:32 * (g + 1)]
        if with_res:
            res_ref[...] = hb @ rw_ref[...] + rb_ref[...]

    in_specs = [_row_spec(din), _full_spec((din, dout)),
                _full_spec((dout, 16)), _full_spec((dout, 16))]
    out_shapes = [jax.ShapeDtypeStruct((G, N, 32), jnp.float32),
                  jax.ShapeDtypeStruct((N, 16), jnp.float32),
                  jax.ShapeDtypeStruct((N, 16), jnp.float32)]
    out_specs = [pl.BlockSpec((G, _RB, 32), lambda i: (0, i, 0)),
                 _row_spec(16), _row_spec(16)]
    if with_res:
        in_specs += [_full_spec((din, dout)), _full_spec((1, dout))]
        out_shapes.append(jax.ShapeDtypeStruct((N, dout), jnp.float32))
        out_specs.append(_row_spec(dout))
    return pl.pallas_call(
        body,
        grid=(_GRID,),
        in_specs=in_specs,
        out_specs=out_specs,
        out_shape=out_shapes,
    )


def _make_post(dout):
    G = dout // 32
    C = dout // HEADS

    def body(outg_ref, s_ref, als_ref, ald_ref, hpg_ref, res_ref,
             bias_ref, ng_ref, nb_ref, out_ref):
        un = jnp.concatenate([outg_ref[g] for g in range(G)], axis=1)
        hp = jnp.concatenate([hpg_ref[g] for g in range(G)], axis=1)
        es = als_ref[...][:, :HEADS] + ald_ref[...][:, :HEADS]
        es = jnp.where(es > 0, es, 0.2 * es)
        exs = jnp.exp(es)
        stot = s_ref[0][:, :HEADS] + s_ref[1][:, :HEADS] + exs
        parts = []
        for h in range(HEADS):
            lo = h * C
            num = (un[:, lo:lo + C]
                   + exs[:, h:h + 1] * hp[:, lo:lo + C])
            parts.append(num / stot[:, h:h + 1])
        gat = jnp.concatenate(parts, axis=1) + bias_ref[...]
        gat = _layer_norm(gat, ng_ref[...], nb_ref[...])
        out_ref[...] = _gelu(gat + res_ref[...])

    return pl.pallas_call(
        body,
        grid=(_GRID,),
        in_specs=[pl.BlockSpec((G, _RB, 32), lambda i: (0, i, 0)),
                  pl.BlockSpec((2, _RB, 16), lambda i: (0, i, 0)),
                  _row_spec(16), _row_spec(16),
                  pl.BlockSpec((G, _RB, 32), lambda i: (0, i, 0)),
                  _row_spec(dout),
                  _full_spec((1, dout)), _full_spec((1, dout)),
                  _full_spec((1, dout))],
        out_specs=_row_spec(dout),
        out_shape=jax.ShapeDtypeStruct((N, dout), jnp.float32),
    )


def _pool_kernel(h_ref, b_ref, sum_ref, max_ref, cnt_ref):
    i = pl.program_id(0)
    hb = h_ref[...]
    seg = lax.broadcasted_iota(jnp.int32, (_RB, 32), 1)
    mask = b_ref[...] == seg
    maskf = mask.astype(jnp.float32)
    dims = (((0,), (0,)), ((), ()))
    psum = lax.dot_general(maskf, hb, dims, precision=lax.Precision.HIGHEST)
    pcnt = lax.dot_general(maskf, jnp.ones_like(hb), dims, precision=lax.Precision.HIGHEST)
    pms = []
    for s in range(32):
        hv = jnp.where(mask[:, s:s + 1], hb, -jnp.inf)
        pms.append(jnp.max(hv, axis=0, keepdims=True))
    pmax = jnp.concatenate(pms, axis=0)

    @pl.when(i == 0)
    def _():
        sum_ref[...] = psum
        max_ref[...] = pmax
        cnt_ref[...] = pcnt

    @pl.when(i != 0)
    def _():
        sum_ref[...] += psum
        max_ref[...] = jnp.maximum(max_ref[...], pmax)
        cnt_ref[...] += pcnt


_pool = pl.pallas_call(
    _pool_kernel,
    grid=(_GRID,),
    in_specs=[_row_spec(128), _row_spec(1)],
    out_specs=[_full_spec((32, 128))] * 3,
    out_shape=[jax.ShapeDtypeStruct((32, 128), jnp.float32)] * 3,
)


def _mlp_kernel(gsum_ref, gmax_ref, cnt_ref, u_ref,
                gw1_ref, gb1_ref, gg1_ref, gbe1_ref,
                gw2_ref, gb2_ref, gg2_ref, gbe2_ref,
                fw1_ref, fb1_ref, fg1_ref, fbe1_ref,
                fw2_ref, fb2_ref, fg2_ref, fbe2_ref,
                cw1_ref, cb1_ref, cw2_ref, cb2_ref, out_ref):
    gsum = gsum_ref[...]
    gmean = gsum / jnp.maximum(cnt_ref[...], 1.0)
    gfeat = jnp.concatenate([gmean, gmax_ref[...], gsum / 10.0], axis=1)
    u = u_ref[...]
    g = _gelu(_layer_norm(u @ gw1_ref[...] + gb1_ref[...], gg1_ref[...],
                          gbe1_ref[...]))
    g = _gelu(_layer_norm(g @ gw2_ref[...] + gb2_ref[...], gg2_ref[...],
                          gbe2_ref[...]))
    c = jnp.concatenate([gfeat, g], axis=1)
    c = _gelu(_layer_norm(c @ fw1_ref[...] + fb1_ref[...], fg1_ref[...],
                          fbe1_ref[...]))
    c = _gelu(_layer_norm(c @ fw2_ref[...] + fb2_ref[...], fg2_ref[...],
                          fbe2_ref[...]))
    c = _gelu(c @ cw1_ref[...] + cb1_ref[...])
    out_ref[...] = c @ cw2_ref[...] + cb2_ref[...]


_enc = _make_enc(16, 64)
_pre_128 = _make_pre(64, 128, True)
_pre_256r = _make_pre(128, 256, True)
_pre_256 = _make_pre(256, 256, False)
_pre_128b = _make_pre(256, 128, True)
_post_128 = _make_post(128)
_post_256 = _make_post(256)


def _attn_mat(a, dout):
    heads, C = a.shape
    eye = jnp.eye(heads, 16, dtype=jnp.float32)
    return (a[:, :, None] * eye[:, None, :]).reshape(dout, 16)


def _gat_block_sc(h, src, dst, n, p, pre, post, zeros16, zeros32):
    D = p['W'].shape[1]
    G = D // 32
    As16 = _attn_mat(p['as'], D)
    Ad16 = _attn_mat(p['ad'], D)
    if 'rw' in p:
        hp_g, als16, ald16, res = pre(h, p['W'], As16, Ad16, p['rw'],
                                      p['rb'][None])
    else:
        hp_g, als16, ald16 = pre(h, p['W'], As16, Ad16)
        res = h
    ex, s_part = _phase1(als16, ald16, src, dst, zeros16)
    phase2 = _phase2_4 if G == 4 else _phase2_8
    out_g = phase2(hp_g.reshape(G * n, 32), src, dst, ex, zeros32)
    return post(out_g, s_part, als16, ald16, hp_g, res,
                p['b'][None], p['ng'][None], p['nb'][None])


def kernel(x, u, params, edge_index, batch):
    n = x.shape[0]
    src = edge_index[0]
    dst = edge_index[1]
    p = params
    zeros16 = jnp.zeros((n, 16), jnp.float32)
    zeros32 = jnp.zeros((n, 32), jnp.float32)

    h = _enc(x, p['ne_w'], p['ne_b'][None], p['ne_g'][None], p['ne_be'][None])
    for name, pre, post in (('g1', _pre_128, _post_128),
                            ('g2', _pre_256r, _post_256),
                            ('g3', _pre_256, _post_256),
                            ('g4', _pre_128b, _post_128)):
        h = _gat_block_sc(h, src, dst, n, p[name], pre, post,
                          zeros16, zeros32)

    gsum, gmax, cnt = _pool(h, batch[:, None].astype(jnp.int32))

    out = pl.pallas_call(
        _mlp_kernel,
        out_shape=jax.ShapeDtypeStruct((32, 1), jnp.float32),
    )(gsum, gmax, cnt, u,
      p['ge_w1'], p['ge_b1'], p['ge_g1'], p['ge_be1'],
      p['ge_w2'], p['ge_b2'], p['ge_g2'], p['ge_be2'],
      p['fu_w1'], p['fu_b1'], p['fu_g1'], p['fu_be1'],
      p['fu_w2'], p['fu_b2'], p['fu_g2'], p['fu_be2'],
      p['cl_w1'], p['cl_b1'], p['cl_w2'], p['cl_b2'])
    return out[:, 0]
